# Initial kernel scaffold; baseline (speedup 1.0000x reference)
import jax
import jax.numpy as jnp
from jax.experimental import pallas as pl


def kernel(x, gate_w, gate_b, expert_w, expert_b):
    raise NotImplementedError("write your pallas kernel here")


# speedup vs baseline: 51.2080x; 51.2080x over previous
import functools

import jax
import jax.numpy as jnp
from jax import lax
from jax.experimental import pallas as pl
from jax.experimental.pallas import tpu as pltpu

D_MODEL = 2048
NUM_EXPERTS = 8
TOP_K = 2


def _gate_body(x_ref, gw_ref, gb_ref, g_ref):
    logits = jnp.dot(gw_ref[...], x_ref[...].T,
                     preferred_element_type=jnp.float32) + gb_ref[...]
    scores = jax.nn.softmax(logits, axis=0)
    iota = lax.broadcasted_iota(jnp.int32, scores.shape, 0)
    big = jnp.int32(NUM_EXPERTS)
    m1 = jnp.max(scores, axis=0, keepdims=True)
    i1 = jnp.min(jnp.where(scores == m1, iota, big), axis=0, keepdims=True)
    mask1 = iota == i1
    s2 = jnp.where(mask1, -jnp.inf, scores)
    m2 = jnp.max(s2, axis=0, keepdims=True)
    i2 = jnp.min(jnp.where(s2 == m2, iota, big), axis=0, keepdims=True)
    mask2 = iota == i2
    g_ref[...] = jnp.where(mask1 | mask2, scores, 0.0)


def _expert_body(x_ref, g_ref, w_ref, b_ref, out_ref):
    e = pl.program_id(1)

    @pl.when(e == 0)
    def _():
        out_ref[...] = jnp.zeros_like(out_ref)

    y = jnp.dot(x_ref[...], w_ref[0].T, preferred_element_type=jnp.float32)
    gcol = g_ref[...].reshape(-1, 1)
    out_ref[...] += gcol * (y + b_ref[0])


def _moe_2d(x2d, gate_w, gate_b, expert_w, expert_b):
    S = x2d.shape[0]
    g = pl.pallas_call(
        _gate_body,
        out_shape=jax.ShapeDtypeStruct((NUM_EXPERTS, S), jnp.float32),
    )(x2d, gate_w, gate_b.reshape(NUM_EXPERTS, 1))

    NO = 512
    grid = (D_MODEL // NO, NUM_EXPERTS)
    out = pl.pallas_call(
        _expert_body,
        grid=grid,
        in_specs=[
            pl.BlockSpec((S, D_MODEL), lambda o, e: (0, 0)),
            pl.BlockSpec((1, 1, S), lambda o, e: (e, 0, 0)),
            pl.BlockSpec((1, NO, D_MODEL), lambda o, e: (e, o, 0)),
            pl.BlockSpec((1, 1, NO), lambda o, e: (e, 0, o)),
        ],
        out_specs=pl.BlockSpec((S, NO), lambda o, e: (0, o)),
        out_shape=jax.ShapeDtypeStruct((S, D_MODEL), jnp.float32),
    )(x2d, g.reshape(NUM_EXPERTS, 1, S), expert_w,
      expert_b.reshape(NUM_EXPERTS, 1, D_MODEL))
    return out


def kernel(x, gate_w, gate_b, expert_w, expert_b):
    B, S, D = x.shape
    out = _moe_2d(x.reshape(B * S, D), gate_w, gate_b, expert_w, expert_b)
    return out.reshape(B, S, D)
